# baseline (speedup 1.0000x reference)
import functools

import jax
import jax.numpy as jnp
from jax import lax
from jax.experimental import pallas as pl
from jax.experimental.pallas import tpu as pltpu
from jax.experimental.pallas import tpu_sc as plsc

N = 10000
E = 320000
D = 128
DCTX = 16
NC = 2
NS = 16
NW = NC * NS
BATCH = 128
NB = 80
CH = 40
EPW = NB * BATCH
EPAD = NW * EPW
NACC = 10240
RPT = NACC // NS
EPS = 1e-5

_mesh = plsc.VectorSubcoreMesh(
    core_axis_name="c", subcore_axis_name="s", num_cores=NC, num_subcores=NS
)


@functools.partial(
    pl.kernel,
    out_type=jax.ShapeDtypeStruct((NC, NACC, D), jnp.float32),
    mesh=_mesh,
    scratch_types=[
        pltpu.VMEM((NB, BATCH), jnp.int32),
        pltpu.VMEM((BATCH, D), jnp.float32),
        pltpu.VMEM_SHARED((NACC, D), jnp.float32),
    ],
)
def _deg_kernel(dst_hbm, zeros_hbm, ones_hbm, out_hbm, idx_v, ones_v, acc_sh):
    c = lax.axis_index("c")
    s = lax.axis_index("s")
    wid = s * NC + c
    pltpu.sync_copy(zeros_hbm.at[pl.ds(s * RPT, RPT)], acc_sh.at[pl.ds(s * RPT, RPT)])
    pltpu.sync_copy(ones_hbm, ones_v)
    pltpu.sync_copy(dst_hbm.at[wid], idx_v)
    plsc.subcore_barrier()

    def body(j, carry):
        pltpu.sync_copy(ones_v, acc_sh.at[idx_v.at[j]], add=True)
        return carry

    lax.fori_loop(0, NB, body, 0)
    plsc.subcore_barrier()
    pltpu.sync_copy(acc_sh.at[pl.ds(s * RPT, RPT)], out_hbm.at[c, pl.ds(s * RPT, RPT)])


@functools.partial(
    pl.kernel,
    out_type=jax.ShapeDtypeStruct((NC, NACC, D), jnp.float32),
    mesh=_mesh,
    scratch_types=[
        pltpu.VMEM((CH, BATCH), jnp.int32),
        pltpu.VMEM((CH, BATCH), jnp.int32),
        pltpu.VMEM((2, BATCH, D), jnp.float32),
        pltpu.VMEM_SHARED((NACC, D), jnp.float32),
        pltpu.SemaphoreType.DMA,
        pltpu.SemaphoreType.DMA,
    ],
)
def _scatter_kernel(h_hbm, src_hbm, dst_hbm, zeros_hbm, out_hbm,
                    src_v, dst_v, rows_v, acc_sh, gsem, ssem):
    c = lax.axis_index("c")
    s = lax.axis_index("s")
    wid = s * NC + c
    pltpu.sync_copy(zeros_hbm.at[pl.ds(s * RPT, RPT)], acc_sh.at[pl.ds(s * RPT, RPT)])
    plsc.subcore_barrier()

    for ci in range(NB // CH):
        pltpu.sync_copy(src_hbm.at[wid, pl.ds(ci * CH, CH)], src_v)
        pltpu.sync_copy(dst_hbm.at[wid, pl.ds(ci * CH, CH)], dst_v)
        pltpu.async_copy(h_hbm.at[src_v.at[0]], rows_v.at[0], gsem)

        def body(j, carry):
            @pl.when(j >= 1)
            def _wait_prev_scatter():
                pltpu.make_async_copy(
                    rows_v.at[(j - 1) % 2], acc_sh.at[dst_v.at[j - 1]], ssem
                ).wait()

            @pl.when(j + 1 < CH)
            def _start_next():
                pltpu.async_copy(h_hbm.at[src_v.at[j + 1]], rows_v.at[(j + 1) % 2], gsem)

            pltpu.make_async_copy(h_hbm.at[src_v.at[j]], rows_v.at[j % 2], gsem).wait()
            pltpu.async_copy(rows_v.at[j % 2], acc_sh.at[dst_v.at[j]], ssem, add=True)
            return carry

        lax.fori_loop(0, CH, body, 0)
        pltpu.make_async_copy(
            rows_v.at[(CH - 1) % 2], acc_sh.at[dst_v.at[CH - 1]], ssem
        ).wait()
    plsc.subcore_barrier()
    pltpu.sync_copy(acc_sh.at[pl.ds(s * RPT, RPT)], out_hbm.at[c, pl.ds(s * RPT, RPT)])


def _tc_mm1_body(x_ref, ctx_ref, w1a_ref, w1b_ref, h_ref):
    cvec = jnp.dot(ctx_ref[...], w1b_ref[...], preferred_element_type=jnp.float32)
    h_ref[...] = jnp.dot(x_ref[...], w1a_ref[...],
                         preferred_element_type=jnp.float32) + cvec


_tc_mm1 = pl.pallas_call(
    _tc_mm1_body,
    out_shape=jax.ShapeDtypeStruct((N, D), jnp.float32),
)


def _tc_scale_body(h_ref, d0_ref, d1_ref, hp_ref, dinv_ref):
    dinv = lax.rsqrt(d0_ref[...] + d1_ref[...] + 1.0)
    hp_ref[...] = h_ref[...] * dinv
    dinv_ref[...] = dinv


_tc_scale = pl.pallas_call(
    _tc_scale_body,
    out_shape=(
        jax.ShapeDtypeStruct((N, D), jnp.float32),
        jax.ShapeDtypeStruct((N, 1), jnp.float32),
    ),
)


def _tc_mid_body(a0_ref, a1_ref, hp_ref, dinv_ref, b_ref, g_ref, be_ref, w2_ref, o_ref):
    dinv = dinv_ref[...]
    m = (a0_ref[...] + a1_ref[...] + hp_ref[...]) * dinv + b_ref[...]
    r = jnp.maximum(m, 0.0)
    mu = jnp.mean(r, axis=0, keepdims=True)
    var = jnp.mean((r - mu) ** 2, axis=0, keepdims=True)
    y = (r - mu) * lax.rsqrt(var + EPS) * g_ref[...] + be_ref[...]
    y = jnp.maximum(y, 0.0)
    o_ref[...] = jnp.dot(y, w2_ref[...], preferred_element_type=jnp.float32) * dinv


_tc_mid = pl.pallas_call(
    _tc_mid_body,
    out_shape=jax.ShapeDtypeStruct((N, D), jnp.float32),
)


def _tc_out_body(a0_ref, a1_ref, hp_ref, dinv_ref, b_ref, g_ref, be_ref,
                 wo_ref, bo_ref, o_ref):
    dinv = dinv_ref[...]
    m = (a0_ref[...] + a1_ref[...] + hp_ref[...]) * dinv + b_ref[...]
    r = jnp.maximum(m, 0.0)
    mu = jnp.mean(r, axis=0, keepdims=True)
    var = jnp.mean((r - mu) ** 2, axis=0, keepdims=True)
    y = (r - mu) * lax.rsqrt(var + EPS) * g_ref[...] + be_ref[...]
    y = jnp.maximum(y, 0.0)
    o_ref[...] = jnp.dot(y, wo_ref[...], preferred_element_type=jnp.float32) + bo_ref[...]


def kernel(x, edge_index, scene_context, W1, b1, g1, be1, W2, b2, g2, be2, Wo, bo):
    src = edge_index[0].astype(jnp.int32)
    dst = edge_index[1].astype(jnp.int32)
    pad = EPAD - E
    pad_src = jnp.arange(pad, dtype=jnp.int32) % N
    pad_dst = N + jnp.arange(pad, dtype=jnp.int32) % (NACC - N)
    srcp = jnp.concatenate([src, pad_src]).reshape(NW, NB, BATCH)
    dstp = jnp.concatenate([dst, pad_dst]).reshape(NW, NB, BATCH)
    zeros_d = jnp.zeros((NACC, D), jnp.float32)
    ones_d = jnp.ones((BATCH, D), jnp.float32)

    h1u = _tc_mm1(x, scene_context.reshape(1, DCTX), W1[:D], W1[D:])
    degp = _deg_kernel(dstp, zeros_d, ones_d)
    d0 = degp[0, :N, 0:1]
    d1 = degp[1, :N, 0:1]

    h1p, dinv = _tc_scale(h1u, d0, d1)

    acc = _scatter_kernel(h1p, srcp, dstp, zeros_d)
    h2p = _tc_mid(acc[0, :N], acc[1, :N], h1p, dinv,
                  b1.reshape(1, D), g1.reshape(1, D), be1.reshape(1, D), W2)

    acc2 = _scatter_kernel(h2p, srcp, dstp, zeros_d)
    nclass = Wo.shape[1]
    tc_out = pl.pallas_call(
        _tc_out_body,
        out_shape=jax.ShapeDtypeStruct((N, nclass), jnp.float32),
    )
    return tc_out(acc2[0, :N], acc2[1, :N], h2p, dinv,
                  b2.reshape(1, D), g2.reshape(1, D), be2.reshape(1, D),
                  Wo, bo.reshape(1, nclass))


# speedup vs baseline: 1.0489x; 1.0489x over previous
import functools

import jax
import jax.numpy as jnp
from jax import lax
from jax.experimental import pallas as pl
from jax.experimental.pallas import tpu as pltpu
from jax.experimental.pallas import tpu_sc as plsc

N = 10000
E = 320000
D = 128
DCTX = 16
NC = 2
NS = 16
NW = NC * NS
BATCH = 128
NB = 80
CH = 40
EPW = NB * BATCH
EPAD = NW * EPW
NACC = 10240
RPT = NACC // NS
NPAD = NACC - N
DEGC = 8
EPS = 1e-5

_mesh = plsc.VectorSubcoreMesh(
    core_axis_name="c", subcore_axis_name="s", num_cores=NC, num_subcores=NS
)


@functools.partial(
    pl.kernel,
    out_type=jax.ShapeDtypeStruct((NC, NACC, D), jnp.float32),
    mesh=_mesh,
    scratch_types=[
        pltpu.VMEM((NB, BATCH), jnp.int32),
        pltpu.VMEM((BATCH, D), jnp.float32),
        pltpu.VMEM_SHARED((NACC, D), jnp.float32),
    ],
)
def _deg_kernel(dst_hbm, zeros_hbm, ones_hbm, out_hbm, idx_v, ones_v, acc_sh):
    c = lax.axis_index("c")
    s = lax.axis_index("s")
    wid = s * NC + c
    pltpu.sync_copy(zeros_hbm.at[pl.ds(s * RPT, RPT)], acc_sh.at[pl.ds(s * RPT, RPT)])
    pltpu.sync_copy(ones_hbm, ones_v)
    pltpu.sync_copy(dst_hbm.at[wid], idx_v)
    plsc.subcore_barrier()

    def body(j, carry):
        pltpu.sync_copy(ones_v, acc_sh.at[idx_v.at[j]], add=True)
        return carry

    lax.fori_loop(0, NB, body, 0)
    plsc.subcore_barrier()
    pltpu.sync_copy(acc_sh.at[pl.ds(s * RPT, RPT)], out_hbm.at[c, pl.ds(s * RPT, RPT)])


@functools.partial(
    pl.kernel,
    out_type=jax.ShapeDtypeStruct((NC, NACC, D), jnp.float32),
    mesh=_mesh,
    scratch_types=[
        pltpu.VMEM((CH, BATCH), jnp.int32),
        pltpu.VMEM((CH, BATCH), jnp.int32),
        pltpu.VMEM((2, BATCH, D), jnp.float32),
        pltpu.VMEM_SHARED((NACC, D), jnp.float32),
        pltpu.SemaphoreType.DMA,
        pltpu.SemaphoreType.DMA,
    ],
)
def _scatter_kernel(h_hbm, src_hbm, dst_hbm, zeros_hbm, out_hbm,
                    src_v, dst_v, rows_v, acc_sh, gsem, ssem):
    c = lax.axis_index("c")
    s = lax.axis_index("s")
    wid = s * NC + c

    @pl.when(c == 0)
    def _seed_h():
        pltpu.sync_copy(h_hbm.at[pl.ds(s * RPT, RPT)], acc_sh.at[pl.ds(s * RPT, RPT)])

    @pl.when(c != 0)
    def _seed_zero():
        pltpu.sync_copy(zeros_hbm.at[pl.ds(s * RPT, RPT)], acc_sh.at[pl.ds(s * RPT, RPT)])

    plsc.subcore_barrier()

    for ci in range(NB // CH):
        pltpu.sync_copy(src_hbm.at[wid, pl.ds(ci * CH, CH)], src_v)
        pltpu.sync_copy(dst_hbm.at[wid, pl.ds(ci * CH, CH)], dst_v)
        pltpu.async_copy(h_hbm.at[src_v.at[0]], rows_v.at[0], gsem)

        def body(j, carry):
            @pl.when(j >= 1)
            def _wait_prev_scatter():
                pltpu.make_async_copy(
                    rows_v.at[(j - 1) % 2], acc_sh.at[dst_v.at[j - 1]], ssem
                ).wait()

            @pl.when(j + 1 < CH)
            def _start_next():
                pltpu.async_copy(h_hbm.at[src_v.at[j + 1]], rows_v.at[(j + 1) % 2], gsem)

            pltpu.make_async_copy(h_hbm.at[src_v.at[j]], rows_v.at[j % 2], gsem).wait()
            pltpu.async_copy(rows_v.at[j % 2], acc_sh.at[dst_v.at[j]], ssem, add=True)
            return carry

        lax.fori_loop(0, CH, body, 0)
        pltpu.make_async_copy(
            rows_v.at[(CH - 1) % 2], acc_sh.at[dst_v.at[CH - 1]], ssem
        ).wait()
    plsc.subcore_barrier()
    pltpu.sync_copy(acc_sh.at[pl.ds(s * RPT, RPT)], out_hbm.at[c, pl.ds(s * RPT, RPT)])


def _tc1_body(x_ref, ctx_ref, w1a_ref, w1b_ref, d0_ref, d1_ref, hp_ref, dinv_ref):
    deg = d0_ref[...] + d1_ref[...] + 1.0
    dinv = lax.rsqrt(deg)
    cvec = jnp.dot(ctx_ref[...], w1b_ref[...], preferred_element_type=jnp.float32)
    h = jnp.dot(x_ref[...], w1a_ref[...], preferred_element_type=jnp.float32) + cvec
    hp_ref[pl.ds(0, N), :] = h * dinv
    hp_ref[pl.ds(N, NPAD), :] = jnp.zeros((NPAD, D), jnp.float32)
    dinv_ref[...] = dinv


_tc1 = pl.pallas_call(
    _tc1_body,
    out_shape=(
        jax.ShapeDtypeStruct((NACC, D), jnp.float32),
        jax.ShapeDtypeStruct((N, 1), jnp.float32),
    ),
)


def _tc_mid_body(a_ref, dinv_ref, b_ref, g_ref, be_ref, w2_ref, o_ref):
    dinv = dinv_ref[...]
    m = (a_ref[0, :N, :] + a_ref[1, :N, :]) * dinv + b_ref[...]
    r = jnp.maximum(m, 0.0)
    mu = jnp.mean(r, axis=0, keepdims=True)
    var = jnp.mean((r - mu) ** 2, axis=0, keepdims=True)
    y = (r - mu) * lax.rsqrt(var + EPS) * g_ref[...] + be_ref[...]
    y = jnp.maximum(y, 0.0)
    o_ref[pl.ds(0, N), :] = jnp.dot(y, w2_ref[...], preferred_element_type=jnp.float32) * dinv
    o_ref[pl.ds(N, NPAD), :] = jnp.zeros((NPAD, D), jnp.float32)


_tc_mid = pl.pallas_call(
    _tc_mid_body,
    out_shape=jax.ShapeDtypeStruct((NACC, D), jnp.float32),
)


def _tc_out_body(a_ref, dinv_ref, b_ref, g_ref, be_ref, wo_ref, bo_ref, o_ref):
    dinv = dinv_ref[...]
    m = (a_ref[0, :N, :] + a_ref[1, :N, :]) * dinv + b_ref[...]
    r = jnp.maximum(m, 0.0)
    mu = jnp.mean(r, axis=0, keepdims=True)
    var = jnp.mean((r - mu) ** 2, axis=0, keepdims=True)
    y = (r - mu) * lax.rsqrt(var + EPS) * g_ref[...] + be_ref[...]
    y = jnp.maximum(y, 0.0)
    o_ref[...] = jnp.dot(y, wo_ref[...], preferred_element_type=jnp.float32) + bo_ref[...]


def kernel(x, edge_index, scene_context, W1, b1, g1, be1, W2, b2, g2, be2, Wo, bo):
    src = edge_index[0].astype(jnp.int32)
    dst = edge_index[1].astype(jnp.int32)
    pad = EPAD - E
    pad_src = jnp.arange(pad, dtype=jnp.int32) % N
    pad_dst = N + jnp.arange(pad, dtype=jnp.int32) % NPAD
    srcp = jnp.concatenate([src, pad_src]).reshape(NW, NB, BATCH)
    dstp = jnp.concatenate([dst, pad_dst]).reshape(NW, NB, BATCH)
    zeros_d = jnp.zeros((NACC, D), jnp.float32)
    ones_d = jnp.ones((BATCH, D), jnp.float32)

    degp = _deg_kernel(dstp, zeros_d, ones_d)
    d0 = degp[0, :N, 0:1]
    d1 = degp[1, :N, 0:1]

    h1p, dinv = _tc1(x, scene_context.reshape(1, DCTX), W1[:D], W1[D:], d0, d1)

    acc = _scatter_kernel(h1p, srcp, dstp, zeros_d)
    h2p = _tc_mid(acc, dinv, b1.reshape(1, D), g1.reshape(1, D), be1.reshape(1, D), W2)

    acc2 = _scatter_kernel(h2p, srcp, dstp, zeros_d)
    nclass = Wo.shape[1]
    tc_out = pl.pallas_call(
        _tc_out_body,
        out_shape=jax.ShapeDtypeStruct((N, nclass), jnp.float32),
    )
    return tc_out(acc2, dinv,
                  b2.reshape(1, D), g2.reshape(1, D), be2.reshape(1, D),
                  Wo, bo.reshape(1, nclass))


# speedup vs baseline: 1.0570x; 1.0077x over previous
import functools

import jax
import jax.numpy as jnp
from jax import lax
from jax.experimental import pallas as pl
from jax.experimental.pallas import tpu as pltpu
from jax.experimental.pallas import tpu_sc as plsc

N = 10000
E = 320000
D = 128
DCTX = 16
NC = 2
NS = 16
NW = NC * NS
BATCH = 128
NB = 80
CH = 40
EPW = NB * BATCH
EPAD = NW * EPW
NACC = 10240
RPT = NACC // NS
NPAD = NACC - N
DEGC = 8
EPS = 1e-5

_mesh = plsc.VectorSubcoreMesh(
    core_axis_name="c", subcore_axis_name="s", num_cores=NC, num_subcores=NS
)


@functools.partial(
    pl.kernel,
    out_type=jax.ShapeDtypeStruct((NC, NACC, D), jnp.float32),
    mesh=_mesh,
    scratch_types=[
        pltpu.VMEM((NB, BATCH), jnp.int32),
        pltpu.VMEM((BATCH, D), jnp.float32),
        pltpu.VMEM_SHARED((NACC, D), jnp.float32),
    ],
)
def _deg_kernel(dst_hbm, zeros_hbm, ones_hbm, out_hbm, idx_v, ones_v, acc_sh):
    c = lax.axis_index("c")
    s = lax.axis_index("s")
    wid = s * NC + c
    pltpu.sync_copy(zeros_hbm.at[pl.ds(s * RPT, RPT)], acc_sh.at[pl.ds(s * RPT, RPT)])
    pltpu.sync_copy(ones_hbm, ones_v)
    pltpu.sync_copy(dst_hbm.at[pl.ds(wid * NB, NB)], idx_v)
    plsc.subcore_barrier()

    def body(j, carry):
        pltpu.sync_copy(ones_v, acc_sh.at[idx_v.at[j]], add=True)
        return carry

    lax.fori_loop(0, NB, body, 0)
    plsc.subcore_barrier()
    pltpu.sync_copy(acc_sh.at[pl.ds(s * RPT, RPT)], out_hbm.at[c, pl.ds(s * RPT, RPT)])


@functools.partial(
    pl.kernel,
    out_type=jax.ShapeDtypeStruct((NC, NACC, D), jnp.float32),
    mesh=_mesh,
    scratch_types=[
        pltpu.VMEM((CH, BATCH), jnp.int32),
        pltpu.VMEM((CH, BATCH), jnp.int32),
        pltpu.VMEM((2, BATCH, D), jnp.float32),
        pltpu.VMEM_SHARED((NACC, D), jnp.float32),
        pltpu.SemaphoreType.DMA,
        pltpu.SemaphoreType.DMA,
    ],
)
def _scatter_kernel(h_hbm, src_hbm, dst_hbm, zeros_hbm, out_hbm,
                    src_v, dst_v, rows_v, acc_sh, gsem, ssem):
    c = lax.axis_index("c")
    s = lax.axis_index("s")
    wid = s * NC + c

    @pl.when(c == 0)
    def _seed_h():
        pltpu.async_copy(h_hbm.at[pl.ds(s * RPT, RPT)], acc_sh.at[pl.ds(s * RPT, RPT)], ssem)

    @pl.when(c != 0)
    def _seed_zero():
        pltpu.async_copy(zeros_hbm.at[pl.ds(s * RPT, RPT)], acc_sh.at[pl.ds(s * RPT, RPT)], ssem)

    pltpu.sync_copy(src_hbm.at[pl.ds(wid * NB, CH)], src_v)
    pltpu.sync_copy(dst_hbm.at[pl.ds(wid * NB, CH)], dst_v)
    pltpu.make_async_copy(
        zeros_hbm.at[pl.ds(s * RPT, RPT)], acc_sh.at[pl.ds(s * RPT, RPT)], ssem
    ).wait()
    plsc.subcore_barrier()

    for ci in range(NB // CH):
        @pl.when(ci > 0)
        def _load_idx():
            pltpu.sync_copy(src_hbm.at[pl.ds(wid * NB + ci * CH, CH)], src_v)
            pltpu.sync_copy(dst_hbm.at[pl.ds(wid * NB + ci * CH, CH)], dst_v)
        pltpu.async_copy(h_hbm.at[src_v.at[0]], rows_v.at[0], gsem)

        def body(j, carry):
            @pl.when(j >= 1)
            def _wait_prev_scatter():
                pltpu.make_async_copy(
                    rows_v.at[(j - 1) % 2], acc_sh.at[dst_v.at[j - 1]], ssem
                ).wait()

            @pl.when(j + 1 < CH)
            def _start_next():
                pltpu.async_copy(h_hbm.at[src_v.at[j + 1]], rows_v.at[(j + 1) % 2], gsem)

            pltpu.make_async_copy(h_hbm.at[src_v.at[j]], rows_v.at[j % 2], gsem).wait()
            pltpu.async_copy(rows_v.at[j % 2], acc_sh.at[dst_v.at[j]], ssem, add=True)
            return carry

        lax.fori_loop(0, CH, body, 0)
        pltpu.make_async_copy(
            rows_v.at[(CH - 1) % 2], acc_sh.at[dst_v.at[CH - 1]], ssem
        ).wait()
    plsc.subcore_barrier()
    pltpu.sync_copy(acc_sh.at[pl.ds(s * RPT, RPT)], out_hbm.at[c, pl.ds(s * RPT, RPT)])


def _tc1_body(x_ref, ctx_ref, w1a_ref, w1b_ref, d0_ref, d1_ref, hp_ref, dinv_ref):
    deg = d0_ref[...] + d1_ref[...] + 1.0
    dinv = lax.rsqrt(deg)
    cvec = jnp.dot(ctx_ref[...], w1b_ref[...], preferred_element_type=jnp.float32)
    h = jnp.dot(x_ref[...], w1a_ref[...], preferred_element_type=jnp.float32) + cvec
    hp_ref[pl.ds(0, N), :] = h * dinv
    hp_ref[pl.ds(N, NPAD), :] = jnp.zeros((NPAD, D), jnp.float32)
    dinv_ref[...] = dinv


_tc1 = pl.pallas_call(
    _tc1_body,
    out_shape=(
        jax.ShapeDtypeStruct((NACC, D), jnp.float32),
        jax.ShapeDtypeStruct((N, 1), jnp.float32),
    ),
)


def _tc_mid_body(a_ref, dinv_ref, b_ref, g_ref, be_ref, w2_ref, o_ref):
    dinv = dinv_ref[...]
    m = (a_ref[0, :N, :] + a_ref[1, :N, :]) * dinv + b_ref[...]
    r = jnp.maximum(m, 0.0)
    mu = jnp.mean(r, axis=0, keepdims=True)
    var = jnp.mean((r - mu) ** 2, axis=0, keepdims=True)
    y = (r - mu) * lax.rsqrt(var + EPS) * g_ref[...] + be_ref[...]
    y = jnp.maximum(y, 0.0)
    o_ref[pl.ds(0, N), :] = jnp.dot(y, w2_ref[...], preferred_element_type=jnp.float32) * dinv
    o_ref[pl.ds(N, NPAD), :] = jnp.zeros((NPAD, D), jnp.float32)


_tc_mid = pl.pallas_call(
    _tc_mid_body,
    out_shape=jax.ShapeDtypeStruct((NACC, D), jnp.float32),
)


def _tc_out_body(a_ref, dinv_ref, b_ref, g_ref, be_ref, wo_ref, bo_ref, o_ref):
    dinv = dinv_ref[...]
    m = (a_ref[0, :N, :] + a_ref[1, :N, :]) * dinv + b_ref[...]
    r = jnp.maximum(m, 0.0)
    mu = jnp.mean(r, axis=0, keepdims=True)
    var = jnp.mean((r - mu) ** 2, axis=0, keepdims=True)
    y = (r - mu) * lax.rsqrt(var + EPS) * g_ref[...] + be_ref[...]
    y = jnp.maximum(y, 0.0)
    o_ref[...] = jnp.dot(y, wo_ref[...], preferred_element_type=jnp.float32) + bo_ref[...]


def kernel(x, edge_index, scene_context, W1, b1, g1, be1, W2, b2, g2, be2, Wo, bo):
    src = edge_index[0].astype(jnp.int32)
    dst = edge_index[1].astype(jnp.int32)
    pad = EPAD - E
    pad_src = jnp.arange(pad, dtype=jnp.int32) % N
    pad_dst = N + jnp.arange(pad, dtype=jnp.int32) % NPAD
    srcp = jnp.concatenate([src, pad_src]).reshape(NW * NB, BATCH)
    dstp = jnp.concatenate([dst, pad_dst]).reshape(NW * NB, BATCH)
    zeros_d = jnp.zeros((NACC, D), jnp.float32)
    ones_d = jnp.ones((BATCH, D), jnp.float32)

    degp = _deg_kernel(dstp, zeros_d, ones_d)
    d0 = degp[0, :N, 0:1]
    d1 = degp[1, :N, 0:1]

    h1p, dinv = _tc1(x, scene_context.reshape(1, DCTX), W1[:D], W1[D:], d0, d1)

    acc = _scatter_kernel(h1p, srcp, dstp, zeros_d)
    h2p = _tc_mid(acc, dinv, b1.reshape(1, D), g1.reshape(1, D), be1.reshape(1, D), W2)

    acc2 = _scatter_kernel(h2p, srcp, dstp, zeros_d)
    nclass = Wo.shape[1]
    tc_out = pl.pallas_call(
        _tc_out_body,
        out_shape=jax.ShapeDtypeStruct((N, nclass), jnp.float32),
    )
    return tc_out(acc2, dinv,
                  b2.reshape(1, D), g2.reshape(1, D), be2.reshape(1, D),
                  Wo, bo.reshape(1, nclass))


# speedup vs baseline: 1.0798x; 1.0216x over previous
import functools

import jax
import jax.numpy as jnp
from jax import lax
from jax.experimental import pallas as pl
from jax.experimental.pallas import tpu as pltpu
from jax.experimental.pallas import tpu_sc as plsc

N = 10000
E = 320000
D = 128
DCTX = 16
NC = 2
NS = 16
NW = NC * NS
BATCH = 128
EROWS = E // BATCH
NB = 80
CH = 40
TAIL = EROWS - (NW - 1) * NB
NACC = 10240
RPT = NACC // NS
NPAD = NACC - N
DEG_DT = jnp.float32
EPS = 1e-5

_mesh = plsc.VectorSubcoreMesh(
    core_axis_name="c", subcore_axis_name="s", num_cores=NC, num_subcores=NS
)


@functools.partial(
    pl.kernel,
    out_type=jax.ShapeDtypeStruct((NC, NACC, D), DEG_DT),
    mesh=_mesh,
    scratch_types=[
        pltpu.VMEM((NB, BATCH), jnp.int32),
        pltpu.VMEM((BATCH, D), DEG_DT),
        pltpu.VMEM_SHARED((NACC, D), DEG_DT),
    ],
)
def _deg_kernel(e_hbm, zeros_hbm, ones_hbm, out_hbm, idx_v, ones_v, acc_sh):
    c = lax.axis_index("c")
    s = lax.axis_index("s")
    wid = s * NC + c
    last = wid == NW - 1
    pltpu.sync_copy(zeros_hbm.at[pl.ds(s * RPT, RPT)], acc_sh.at[pl.ds(s * RPT, RPT)])
    pltpu.sync_copy(ones_hbm, ones_v)

    @pl.when(jnp.logical_not(last))
    def _load_main():
        pltpu.sync_copy(e_hbm.at[1, pl.ds(wid * NB, NB)], idx_v)

    @pl.when(last)
    def _load_tail():
        pltpu.sync_copy(e_hbm.at[1, pl.ds((NW - 1) * NB, TAIL)], idx_v.at[pl.ds(0, TAIL)])

    plsc.subcore_barrier()
    nb = jnp.where(last, TAIL, NB)

    def body(j, carry):
        pltpu.sync_copy(ones_v, acc_sh.at[idx_v.at[j]], add=True)
        return carry

    lax.fori_loop(0, nb, body, 0)
    plsc.subcore_barrier()
    pltpu.sync_copy(acc_sh.at[pl.ds(s * RPT, RPT)], out_hbm.at[c, pl.ds(s * RPT, RPT)])


@functools.partial(
    pl.kernel,
    out_type=jax.ShapeDtypeStruct((NC, NACC, D), jnp.float32),
    mesh=_mesh,
    scratch_types=[
        pltpu.VMEM((CH, BATCH), jnp.int32),
        pltpu.VMEM((CH, BATCH), jnp.int32),
        pltpu.VMEM((2, BATCH, D), jnp.float32),
        pltpu.VMEM_SHARED((NACC, D), jnp.float32),
        pltpu.SemaphoreType.DMA,
        pltpu.SemaphoreType.DMA,
    ],
)
def _scatter_kernel(h_hbm, e_hbm, zeros_hbm, out_hbm,
                    src_v, dst_v, rows_v, acc_sh, gsem, ssem):
    c = lax.axis_index("c")
    s = lax.axis_index("s")
    wid = s * NC + c
    last = wid == NW - 1

    @pl.when(c == 0)
    def _seed_h():
        pltpu.async_copy(h_hbm.at[pl.ds(s * RPT, RPT)], acc_sh.at[pl.ds(s * RPT, RPT)], ssem)

    @pl.when(c != 0)
    def _seed_zero():
        pltpu.async_copy(zeros_hbm.at[pl.ds(s * RPT, RPT)], acc_sh.at[pl.ds(s * RPT, RPT)], ssem)

    pltpu.make_async_copy(
        zeros_hbm.at[pl.ds(s * RPT, RPT)], acc_sh.at[pl.ds(s * RPT, RPT)], ssem
    ).wait()
    plsc.subcore_barrier()

    def _pipeline(base, ch):
        pltpu.sync_copy(e_hbm.at[0, pl.ds(base, ch)], src_v.at[pl.ds(0, ch)])
        pltpu.sync_copy(e_hbm.at[1, pl.ds(base, ch)], dst_v.at[pl.ds(0, ch)])
        pltpu.async_copy(h_hbm.at[src_v.at[0]], rows_v.at[0], gsem)

        def body(j, carry):
            @pl.when(j >= 1)
            def _wait_prev_scatter():
                pltpu.make_async_copy(
                    rows_v.at[(j - 1) % 2], acc_sh.at[dst_v.at[j - 1]], ssem
                ).wait()

            @pl.when(j + 1 < ch)
            def _start_next():
                pltpu.async_copy(h_hbm.at[src_v.at[j + 1]], rows_v.at[(j + 1) % 2], gsem)

            pltpu.make_async_copy(h_hbm.at[src_v.at[j]], rows_v.at[j % 2], gsem).wait()
            pltpu.async_copy(rows_v.at[j % 2], acc_sh.at[dst_v.at[j]], ssem, add=True)
            return carry

        lax.fori_loop(0, ch, body, 0)
        pltpu.make_async_copy(
            rows_v.at[(ch - 1) % 2], acc_sh.at[dst_v.at[ch - 1]], ssem
        ).wait()

    @pl.when(jnp.logical_not(last))
    def _main():
        for ci in range(NB // CH):
            _pipeline(wid * NB + ci * CH, CH)

    @pl.when(last)
    def _tail():
        _pipeline((NW - 1) * NB, TAIL)

    plsc.subcore_barrier()
    pltpu.sync_copy(acc_sh.at[pl.ds(s * RPT, RPT)], out_hbm.at[c, pl.ds(s * RPT, RPT)])


def _tc1_body(x_ref, ctx_ref, w1a_ref, w1b_ref, d0_ref, d1_ref, hp_ref, dinv_ref):
    deg = d0_ref[...] + d1_ref[...] + 1.0
    dinv = lax.rsqrt(deg)
    cvec = jnp.dot(ctx_ref[...], w1b_ref[...], preferred_element_type=jnp.float32)
    h = jnp.dot(x_ref[...], w1a_ref[...], preferred_element_type=jnp.float32) + cvec
    hp_ref[pl.ds(0, N), :] = h * dinv
    hp_ref[pl.ds(N, NPAD), :] = jnp.zeros((NPAD, D), jnp.float32)
    dinv_ref[...] = dinv


_tc1 = pl.pallas_call(
    _tc1_body,
    out_shape=(
        jax.ShapeDtypeStruct((NACC, D), jnp.float32),
        jax.ShapeDtypeStruct((N, 1), jnp.float32),
    ),
)


def _tc_mid_body(a_ref, dinv_ref, b_ref, g_ref, be_ref, w2_ref, o_ref):
    dinv = dinv_ref[...]
    m = (a_ref[0, :N, :] + a_ref[1, :N, :]) * dinv + b_ref[...]
    r = jnp.maximum(m, 0.0)
    mu = jnp.mean(r, axis=0, keepdims=True)
    var = jnp.mean((r - mu) ** 2, axis=0, keepdims=True)
    y = (r - mu) * lax.rsqrt(var + EPS) * g_ref[...] + be_ref[...]
    y = jnp.maximum(y, 0.0)
    o_ref[pl.ds(0, N), :] = jnp.dot(y, w2_ref[...], preferred_element_type=jnp.float32) * dinv
    o_ref[pl.ds(N, NPAD), :] = jnp.zeros((NPAD, D), jnp.float32)


_tc_mid = pl.pallas_call(
    _tc_mid_body,
    out_shape=jax.ShapeDtypeStruct((NACC, D), jnp.float32),
)


def _tc_out_body(a_ref, dinv_ref, b_ref, g_ref, be_ref, wo_ref, bo_ref, o_ref):
    dinv = dinv_ref[...]
    m = (a_ref[0, :N, :] + a_ref[1, :N, :]) * dinv + b_ref[...]
    r = jnp.maximum(m, 0.0)
    mu = jnp.mean(r, axis=0, keepdims=True)
    var = jnp.mean((r - mu) ** 2, axis=0, keepdims=True)
    y = (r - mu) * lax.rsqrt(var + EPS) * g_ref[...] + be_ref[...]
    y = jnp.maximum(y, 0.0)
    o_ref[...] = jnp.dot(y, wo_ref[...], preferred_element_type=jnp.float32) + bo_ref[...]


def kernel(x, edge_index, scene_context, W1, b1, g1, be1, W2, b2, g2, be2, Wo, bo):
    edge3 = edge_index.astype(jnp.int32).reshape(2, EROWS, BATCH)
    zeros_d = jnp.zeros((NACC, D), jnp.float32)
    ones_d = jnp.ones((BATCH, D), DEG_DT)

    degp = _deg_kernel(edge3, zeros_d, ones_d)
    d0 = degp[0, :N, 0:1]
    d1 = degp[1, :N, 0:1]

    h1p, dinv = _tc1(x, scene_context.reshape(1, DCTX), W1[:D], W1[D:], d0, d1)

    acc = _scatter_kernel(h1p, edge3, zeros_d)
    h2p = _tc_mid(acc, dinv, b1.reshape(1, D), g1.reshape(1, D), be1.reshape(1, D), W2)

    acc2 = _scatter_kernel(h2p, edge3, zeros_d)
    nclass = Wo.shape[1]
    tc_out = pl.pallas_call(
        _tc_out_body,
        out_shape=jax.ShapeDtypeStruct((N, nclass), jnp.float32),
    )
    return tc_out(acc2, dinv,
                  b2.reshape(1, D), g2.reshape(1, D), be2.reshape(1, D),
                  Wo, bo.reshape(1, nclass))


# speedup vs baseline: 1.1257x; 1.0425x over previous
import functools

import jax
import jax.numpy as jnp
from jax import lax
from jax.experimental import pallas as pl
from jax.experimental.pallas import tpu as pltpu
from jax.experimental.pallas import tpu_sc as plsc

N = 10000
E = 320000
D = 128
DCTX = 16
NC = 2
NS = 16
NW = NC * NS
BATCH = 128
EROWS = E // BATCH
NB = 80
CH = 40
TAIL = EROWS - (NW - 1) * NB
NACC = 10240
RPT = NACC // NS
NPAD = NACC - N
DEG_DT = jnp.float32
EPS = 1e-5

_mesh = plsc.VectorSubcoreMesh(
    core_axis_name="c", subcore_axis_name="s", num_cores=NC, num_subcores=NS
)


@functools.partial(
    pl.kernel,
    out_type=jax.ShapeDtypeStruct((NC, NACC, D), DEG_DT),
    mesh=_mesh,
    scratch_types=[
        pltpu.VMEM((NB, BATCH), jnp.int32),
        pltpu.VMEM((BATCH, D), DEG_DT),
        pltpu.VMEM_SHARED((NACC, D), DEG_DT),
        pltpu.SemaphoreType.DMA,
    ],
)
def _deg_kernel(e_hbm, zeros_hbm, ones_hbm, out_hbm, idx_v, ones_v, acc_sh, zsem):
    c = lax.axis_index("c")
    s = lax.axis_index("s")
    wid = s * NC + c
    last = wid == NW - 1
    pltpu.async_copy(zeros_hbm.at[pl.ds(s * RPT, RPT)], acc_sh.at[pl.ds(s * RPT, RPT)], zsem)
    pltpu.sync_copy(ones_hbm, ones_v)

    @pl.when(jnp.logical_not(last))
    def _load_main():
        pltpu.sync_copy(e_hbm.at[1, pl.ds(wid * NB, NB)], idx_v)

    @pl.when(last)
    def _load_tail():
        pltpu.sync_copy(e_hbm.at[1, pl.ds((NW - 1) * NB, TAIL)], idx_v.at[pl.ds(0, TAIL)])

    pltpu.make_async_copy(
        zeros_hbm.at[pl.ds(s * RPT, RPT)], acc_sh.at[pl.ds(s * RPT, RPT)], zsem
    ).wait()
    plsc.subcore_barrier()
    nb = jnp.where(last, TAIL, NB)

    def body(j, carry):
        pltpu.sync_copy(ones_v, acc_sh.at[idx_v.at[j]], add=True)
        return carry

    lax.fori_loop(0, nb, body, 0)
    plsc.subcore_barrier()
    pltpu.sync_copy(acc_sh.at[pl.ds(s * RPT, RPT)], out_hbm.at[c, pl.ds(s * RPT, RPT)])


@functools.partial(
    pl.kernel,
    out_type=jax.ShapeDtypeStruct((NC, NACC, D), jnp.float32),
    mesh=_mesh,
    scratch_types=[
        pltpu.VMEM((CH, BATCH), jnp.int32),
        pltpu.VMEM((CH, BATCH), jnp.int32),
        pltpu.VMEM((2, BATCH, D), jnp.float32),
        pltpu.VMEM_SHARED((NACC, D), jnp.float32),
        pltpu.SemaphoreType.DMA,
        pltpu.SemaphoreType.DMA,
    ],
)
def _scatter_kernel(h_hbm, e_hbm, zeros_hbm, out_hbm,
                    src_v, dst_v, rows_v, acc_sh, gsem, ssem):
    c = lax.axis_index("c")
    s = lax.axis_index("s")
    wid = s * NC + c
    last = wid == NW - 1

    @pl.when(c == 0)
    def _seed_h():
        pltpu.async_copy(h_hbm.at[pl.ds(s * RPT, RPT)], acc_sh.at[pl.ds(s * RPT, RPT)], ssem)

    @pl.when(c != 0)
    def _seed_zero():
        pltpu.async_copy(zeros_hbm.at[pl.ds(s * RPT, RPT)], acc_sh.at[pl.ds(s * RPT, RPT)], ssem)

    @pl.when(jnp.logical_not(last))
    def _pref_main():
        pltpu.sync_copy(e_hbm.at[0, pl.ds(wid * NB, CH)], src_v)
        pltpu.sync_copy(e_hbm.at[1, pl.ds(wid * NB, CH)], dst_v)

    @pl.when(last)
    def _pref_tail():
        pltpu.sync_copy(e_hbm.at[0, pl.ds((NW - 1) * NB, TAIL)], src_v.at[pl.ds(0, TAIL)])
        pltpu.sync_copy(e_hbm.at[1, pl.ds((NW - 1) * NB, TAIL)], dst_v.at[pl.ds(0, TAIL)])

    pltpu.async_copy(h_hbm.at[src_v.at[0]], rows_v.at[0], gsem)
    pltpu.make_async_copy(
        zeros_hbm.at[pl.ds(s * RPT, RPT)], acc_sh.at[pl.ds(s * RPT, RPT)], ssem
    ).wait()
    plsc.subcore_barrier()

    def _pipeline(base, ch, prefetched=False):
        if not prefetched:
            pltpu.sync_copy(e_hbm.at[0, pl.ds(base, ch)], src_v.at[pl.ds(0, ch)])
            pltpu.sync_copy(e_hbm.at[1, pl.ds(base, ch)], dst_v.at[pl.ds(0, ch)])
            pltpu.async_copy(h_hbm.at[src_v.at[0]], rows_v.at[0], gsem)

        def body(j, carry):
            @pl.when(j >= 1)
            def _wait_prev_scatter():
                pltpu.make_async_copy(
                    rows_v.at[(j - 1) % 2], acc_sh.at[dst_v.at[j - 1]], ssem
                ).wait()

            @pl.when(j + 1 < ch)
            def _start_next():
                pltpu.async_copy(h_hbm.at[src_v.at[j + 1]], rows_v.at[(j + 1) % 2], gsem)

            pltpu.make_async_copy(h_hbm.at[src_v.at[j]], rows_v.at[j % 2], gsem).wait()
            pltpu.async_copy(rows_v.at[j % 2], acc_sh.at[dst_v.at[j]], ssem, add=True)
            return carry

        lax.fori_loop(0, ch, body, 0)
        pltpu.make_async_copy(
            rows_v.at[(ch - 1) % 2], acc_sh.at[dst_v.at[ch - 1]], ssem
        ).wait()

    @pl.when(jnp.logical_not(last))
    def _main():
        for ci in range(NB // CH):
            _pipeline(wid * NB + ci * CH, CH, prefetched=(ci == 0))

    @pl.when(last)
    def _tail():
        _pipeline((NW - 1) * NB, TAIL, prefetched=True)

    plsc.subcore_barrier()
    pltpu.sync_copy(acc_sh.at[pl.ds(s * RPT, RPT)], out_hbm.at[c, pl.ds(s * RPT, RPT)])


def _tc1_body(x_ref, ctx_ref, w1a_ref, w1b_ref, dg_ref, hp_ref, dinv_ref):
    deg = dg_ref[0, :N, 0:1] + dg_ref[1, :N, 0:1] + 1.0
    dinv = lax.rsqrt(deg)
    cvec = jnp.dot(ctx_ref[...], w1b_ref[...], preferred_element_type=jnp.float32)
    h = jnp.dot(x_ref[...], w1a_ref[...], preferred_element_type=jnp.float32) + cvec
    hp_ref[pl.ds(0, N), :] = h * dinv
    hp_ref[pl.ds(N, NPAD), :] = jnp.zeros((NPAD, D), jnp.float32)
    dinv_ref[...] = dinv


_tc1 = pl.pallas_call(
    _tc1_body,
    out_shape=(
        jax.ShapeDtypeStruct((NACC, D), jnp.float32),
        jax.ShapeDtypeStruct((N, 1), jnp.float32),
    ),
)


def _tc_mid_body(a_ref, dinv_ref, b_ref, g_ref, be_ref, w2_ref, o_ref):
    dinv = dinv_ref[...]
    m = (a_ref[0, :N, :] + a_ref[1, :N, :]) * dinv + b_ref[...]
    r = jnp.maximum(m, 0.0)
    mu = jnp.mean(r, axis=0, keepdims=True)
    var = jnp.mean((r - mu) ** 2, axis=0, keepdims=True)
    y = (r - mu) * lax.rsqrt(var + EPS) * g_ref[...] + be_ref[...]
    y = jnp.maximum(y, 0.0)
    o_ref[pl.ds(0, N), :] = jnp.dot(y, w2_ref[...], preferred_element_type=jnp.float32) * dinv
    o_ref[pl.ds(N, NPAD), :] = jnp.zeros((NPAD, D), jnp.float32)


_tc_mid = pl.pallas_call(
    _tc_mid_body,
    out_shape=jax.ShapeDtypeStruct((NACC, D), jnp.float32),
)


def _tc_out_body(a_ref, dinv_ref, b_ref, g_ref, be_ref, wo_ref, bo_ref, o_ref):
    dinv = dinv_ref[...]
    m = (a_ref[0, :N, :] + a_ref[1, :N, :]) * dinv + b_ref[...]
    r = jnp.maximum(m, 0.0)
    mu = jnp.mean(r, axis=0, keepdims=True)
    var = jnp.mean((r - mu) ** 2, axis=0, keepdims=True)
    y = (r - mu) * lax.rsqrt(var + EPS) * g_ref[...] + be_ref[...]
    y = jnp.maximum(y, 0.0)
    o_ref[...] = jnp.dot(y, wo_ref[...], preferred_element_type=jnp.float32) + bo_ref[...]


def kernel(x, edge_index, scene_context, W1, b1, g1, be1, W2, b2, g2, be2, Wo, bo):
    edge3 = edge_index.astype(jnp.int32).reshape(2, EROWS, BATCH)
    zeros_d = jnp.zeros((NACC, D), jnp.float32)
    ones_d = jnp.ones((BATCH, D), DEG_DT)

    degp = _deg_kernel(edge3, zeros_d, ones_d)

    h1p, dinv = _tc1(x, scene_context.reshape(1, DCTX), W1[:D], W1[D:], degp)

    acc = _scatter_kernel(h1p, edge3, zeros_d)
    h2p = _tc_mid(acc, dinv, b1.reshape(1, D), g1.reshape(1, D), be1.reshape(1, D), W2)

    acc2 = _scatter_kernel(h2p, edge3, zeros_d)
    nclass = Wo.shape[1]
    tc_out = pl.pallas_call(
        _tc_out_body,
        out_shape=jax.ShapeDtypeStruct((N, nclass), jnp.float32),
    )
    return tc_out(acc2, dinv,
                  b2.reshape(1, D), g2.reshape(1, D), be2.reshape(1, D),
                  Wo, bo.reshape(1, nclass))
